# baseline (speedup 1.0000x reference)
import functools
import math

import jax
import jax.numpy as jnp
from jax import lax
from jax.experimental import pallas as pl
from jax.experimental.pallas import tpu as pltpu
from jax.experimental.pallas import tpu_sc as plsc

N = 10000
N_PAD = 10240
E = 160000
E_PAD = 163840
F_IN = 5
H = 128
L_OUT = 64
NG = 64
CUTOFF = 10.0
G_TOTAL = 68
PD = 16

NCORES = 2
NSUB = 16
NW = NCORES * NSUB
CHUNK = 128


def _sc_gather_pos(pos_flat, idx):
    B = idx.shape[0]
    per_w = B // NW
    nchunk = per_w // CHUNK
    mesh = plsc.VectorSubcoreMesh(core_axis_name="c", subcore_axis_name="s")

    @functools.partial(
        pl.kernel,
        mesh=mesh,
        out_type=jax.ShapeDtypeStruct((3, B), jnp.float32),
        compiler_params=pltpu.CompilerParams(needs_layout_passes=False),
        scratch_types=[
            pltpu.VMEM((4 * N,), jnp.float32),
            pltpu.VMEM((CHUNK,), jnp.int32),
            pltpu.VMEM((3, CHUNK), jnp.float32),
            pltpu.SemaphoreType.DMA,
        ],
    )
    def k(tab_hbm, idx_hbm, out_hbm, tab_v, idx_v, out_v, sem):
        wid = lax.axis_index("s") * NCORES + lax.axis_index("c")
        base = wid * per_w
        pltpu.sync_copy(tab_hbm, tab_v)

        def body(c, _):
            off = base + c * CHUNK
            pltpu.sync_copy(idx_hbm.at[pl.ds(off, CHUNK)], idx_v)
            for s in range(CHUNK // 16):
                sl = pl.ds(s * 16, 16)
                addr = idx_v[sl] * 4
                for comp in range(3):
                    out_v[comp, sl] = plsc.load_gather(tab_v, [addr + comp])
            pltpu.sync_copy(out_v, out_hbm.at[:, pl.ds(off, CHUNK)])
            return ()

        lax.fori_loop(0, nchunk, body, ())

    return k(pos_flat, idx)


CH2 = 40


def _sc_gather_mul_scatter(hm, g, jj, ii):
    per_w = E_PAD // NW
    nchunk = per_w // CH2
    rows_per_tile = N_PAD // NSUB
    mesh = plsc.VectorSubcoreMesh(core_axis_name="c", subcore_axis_name="s")

    @functools.partial(
        pl.kernel,
        mesh=mesh,
        out_type=jax.ShapeDtypeStruct((NCORES, N_PAD, H), jnp.float32),
        compiler_params=pltpu.CompilerParams(needs_layout_passes=False),
        scratch_types=[
            pltpu.VMEM((2, CH2), jnp.int32),
            pltpu.VMEM((4, CH2), jnp.int32),
            pltpu.VMEM((2, CH2, H), jnp.float32),
            pltpu.VMEM((2, CH2, H), jnp.float32),
            pltpu.VMEM((2, CH2, H), jnp.float32),
            pltpu.VMEM_SHARED((N_PAD, H), jnp.float32),
        ] + [pltpu.SemaphoreType.DMA] * 12,
    )
    def k(hm_hbm, g_hbm, j_hbm, i_hbm, out_hbm, jv, iv, rows_v, g_v, msg_v,
          acc_sh, sj0, sj1, si0, si1, si2, si3, sg0, sg1, sr0, sr1, ss0,
          ss1):
        sj = (sj0, sj1)
        si = (si0, si1, si2, si3)
        sg = (sg0, sg1)
        sr = (sr0, sr1)
        ss = (ss0, ss1)
        cid = lax.axis_index("c")
        sid = lax.axis_index("s")
        wid = sid * NCORES + cid
        base = wid * per_w

        def zrow(r, _):
            for cc in range(H // 16):
                msg_v[0, r, pl.ds(cc * 16, 16)] = jnp.zeros(
                    (16,), jnp.float32)
            return ()

        lax.fori_loop(0, CH2, zrow, ())

        def zcopy(z, _):
            pltpu.sync_copy(
                msg_v.at[0],
                acc_sh.at[pl.ds(sid * rows_per_tile + z * CH2, CH2)])
            return ()

        lax.fori_loop(0, rows_per_tile // CH2, zcopy, ())
        plsc.subcore_barrier()

        def start_loads(c, b, b4):
            off = base + c * CH2
            pltpu.async_copy(j_hbm.at[pl.ds(off, CH2)], jv.at[b], sj[b])
            pltpu.async_copy(i_hbm.at[pl.ds(off, CH2)], iv.at[b4], si[b4])
            pltpu.async_copy(g_hbm.at[pl.ds(off, CH2)], g_v.at[b], sg[b])

        def start_gather(b):
            pltpu.async_copy(hm_hbm.at[jv.at[b]], rows_v.at[b], sr[b])

        def wait(sem, src, dst):
            pltpu.make_async_copy(src, dst, sem).wait()

        start_loads(0, 0, 0)
        start_loads(1, 1, 1)
        wait(sj[0], j_hbm.at[pl.ds(base, CH2)], jv.at[0])
        start_gather(0)

        def body(k4, _):
            for b4 in range(4):
                c = 4 * k4 + b4
                b = b4 % 2
                bn = 1 - b
                b4n = (b4 + 2) % 4

                @pl.when(c + 1 < nchunk)
                def _():
                    wait(sj[bn], j_hbm.at[pl.ds(base, CH2)], jv.at[bn])
                    start_gather(bn)

                @pl.when(c >= 2)
                def _():
                    wait(ss[b], msg_v.at[b], acc_sh.at[iv.at[b4]])

                wait(sr[b], hm_hbm.at[jv.at[b]], rows_v.at[b])
                wait(sg[b], g_hbm.at[pl.ds(base, CH2)], g_v.at[b])

                def mul_row(r, _):
                    for cc in range(H // 16):
                        sl = pl.ds(cc * 16, 16)
                        msg_v[b, r, sl] = rows_v[b, r, sl] * g_v[b, r, sl]
                    return ()

                lax.fori_loop(0, CH2, mul_row, ())

                wait(si[b4], i_hbm.at[pl.ds(base, CH2)], iv.at[b4])
                pltpu.async_copy(msg_v.at[b], acc_sh.at[iv.at[b4]], ss[b],
                                 add=True)

                @pl.when(c + 2 < nchunk)
                def _():
                    start_loads(c + 2, b, b4n)

            return ()

        lax.fori_loop(0, nchunk // 4, body, ())
        wait(ss[0], msg_v.at[0], acc_sh.at[iv.at[0]])
        wait(ss[1], msg_v.at[1], acc_sh.at[iv.at[1]])
        plsc.subcore_barrier()
        pltpu.sync_copy(
            acc_sh.at[pl.ds(sid * rows_per_tile, rows_per_tile)],
            out_hbm.at[cid, pl.ds(sid * rows_per_tile, rows_per_tile)])

    return k(hm, g, jj, ii)


def _ssp(v):
    return jax.nn.softplus(v) - math.log(2.0)


def _gate_body(p_ref, wg1_ref, bg1_ref, wg2_ref, bg2_ref, off_ref, coef_ref,
               g_ref, *, be):
    eps = 1e-8
    p = p_ref[...]

    def comp(a, c):
        return p[c, a, :]

    pix, piy, piz = comp(0, 0), comp(0, 1), comp(0, 2)
    pjx, pjy, pjz = comp(1, 0), comp(1, 1), comp(1, 2)
    pkx, pky, pkz = comp(2, 0), comp(2, 1), comp(2, 2)
    plx, ply, plz = comp(3, 0), comp(3, 1), comp(3, 2)

    b1x, b1y, b1z = pjx - pix, pjy - piy, pjz - piz
    b2x, b2y, b2z = pkx - pjx, pky - pjy, pkz - pjz
    b3x, b3y, b3z = plx - pkx, ply - pky, plz - pkz

    dist = jnp.sqrt(b1x * b1x + b1y * b1y + b1z * b1z + eps)

    dot12 = b1x * b2x + b1y * b2y + b1z * b2z
    n_v1 = jnp.sqrt(b1x * b1x + b1y * b1y + b1z * b1z)
    n_v2 = jnp.sqrt(b2x * b2x + b2y * b2y + b2z * b2z)
    cos_a = -dot12 / (n_v1 * n_v2 + eps)
    cos_a = jnp.clip(cos_a, -1.0 + 1e-7, 1.0 - 1e-7)
    ang = jnp.arctan2(jnp.sqrt(1.0 - cos_a * cos_a), cos_a)

    n1x = b1y * b2z - b1z * b2y
    n1y = b1z * b2x - b1x * b2z
    n1z = b1x * b2y - b1y * b2x
    n2x = b2y * b3z - b2z * b3y
    n2y = b2z * b3x - b2x * b3z
    n2z = b2x * b3y - b2y * b3x
    inv_nb2 = 1.0 / (jnp.sqrt(b2x * b2x + b2y * b2y + b2z * b2z) + eps)
    ux, uy, uz = b2x * inv_nb2, b2y * inv_nb2, b2z * inv_nb2
    m1x = n1y * uz - n1z * uy
    m1y = n1z * ux - n1x * uz
    m1z = n1x * uy - n1y * ux
    yv = m1x * n2x + m1y * n2y + m1z * n2z
    xv = n1x * n2x + n1y * n2y + n1z * n2z
    tor = jnp.arctan2(yv, xv + eps)

    off = off_ref[...]
    coef = coef_ref[...]
    col = lax.broadcasted_iota(jnp.int32, (1, 128), 1)
    val = jnp.where(col < 50, dist[:, None],
                    jnp.where(col < 56, ang[:, None], tor[:, None]))
    dlt = val - off
    rbf = jnp.exp(coef * dlt * dlt) * (col < G_TOTAL).astype(jnp.float32)

    cut = 0.5 * (jnp.cos(dist * (math.pi / CUTOFF)) + 1.0)
    cut = cut * (dist < CUTOFF).astype(jnp.float32)
    row = pl.program_id(0) * be + lax.broadcasted_iota(jnp.int32, (be,), 0)
    cut = cut * (row < E).astype(jnp.float32)

    w1 = wg1_ref[...]
    bb1 = bg1_ref[...]
    w2 = wg2_ref[...]
    bb2 = bg2_ref[...]
    for t in range(3):
        gm = _ssp(
            jnp.dot(rbf, w1[t], preferred_element_type=jnp.float32) + bb1[t])
        gt = _ssp(
            jnp.dot(gm, w2[t], preferred_element_type=jnp.float32) + bb2[t])
        g_ref[t, :, :] = gt * cut[:, None]


def _tc_gates(p4, wg1p, bg1, wg2, bg2, offs, coefs, be):
    grid = (E_PAD // be,)
    return pl.pallas_call(
        functools.partial(_gate_body, be=be),
        grid=grid,
        in_specs=[
            pl.BlockSpec((3, 4, be), lambda e: (0, 0, e)),
            pl.BlockSpec((3, 128, H), lambda e: (0, 0, 0)),
            pl.BlockSpec((3, H), lambda e: (0, 0)),
            pl.BlockSpec((3, H, H), lambda e: (0, 0, 0)),
            pl.BlockSpec((3, H), lambda e: (0, 0)),
            pl.BlockSpec((1, 128), lambda e: (0, 0)),
            pl.BlockSpec((1, 128), lambda e: (0, 0)),
        ],
        out_specs=pl.BlockSpec((3, be, H), lambda e: (0, e, 0)),
        out_shape=jax.ShapeDtypeStruct((3, E_PAD, H), jnp.float32),
    )(p4, wg1p, bg1, wg2, bg2, offs, coefs)


def _h0_body(x_ref, w0_ref, b0_ref, wm_ref, h_ref, hm_ref):
    h = jnp.dot(x_ref[...], w0_ref[...],
                preferred_element_type=jnp.float32) + b0_ref[...]
    h_ref[...] = h
    hm_ref[...] = jnp.dot(h, wm_ref[...], preferred_element_type=jnp.float32)


def _tc_h0(x, w0, b0, wm0, bn):
    grid = (N // bn,)
    return pl.pallas_call(
        _h0_body,
        grid=grid,
        in_specs=[
            pl.BlockSpec((bn, F_IN), lambda n: (n, 0)),
            pl.BlockSpec((F_IN, H), lambda n: (0, 0)),
            pl.BlockSpec((1, H), lambda n: (0, 0)),
            pl.BlockSpec((H, H), lambda n: (0, 0)),
        ],
        out_specs=[
            pl.BlockSpec((bn, H), lambda n: (n, 0)),
            pl.BlockSpec((bn, H), lambda n: (n, 0)),
        ],
        out_shape=[
            jax.ShapeDtypeStruct((N, H), jnp.float32),
            jax.ShapeDtypeStruct((N, H), jnp.float32),
        ],
    )(x, w0, b0.reshape(1, H), wm0)


def _upd_body(h_ref, agg_ref, wu_ref, bu_ref, wn_ref, h_out_ref, hm_out_ref):
    agg = (agg_ref[0].astype(jnp.float32) + agg_ref[1].astype(jnp.float32))
    up = _ssp(
        jnp.dot(agg, wu_ref[...], preferred_element_type=jnp.float32) +
        bu_ref[...])
    h = h_ref[...] + up
    h_out_ref[...] = h
    hm_out_ref[...] = jnp.dot(h, wn_ref[...],
                              preferred_element_type=jnp.float32)


def _tc_update(h, agg2, wu, bu, wnext, bn):
    grid = (N // bn,)
    return pl.pallas_call(
        _upd_body,
        grid=grid,
        in_specs=[
            pl.BlockSpec((bn, H), lambda n: (n, 0)),
            pl.BlockSpec((2, bn, H), lambda n: (0, n, 0)),
            pl.BlockSpec((H, H), lambda n: (0, 0)),
            pl.BlockSpec((1, H), lambda n: (0, 0)),
            pl.BlockSpec((H, H), lambda n: (0, 0)),
        ],
        out_specs=[
            pl.BlockSpec((bn, H), lambda n: (n, 0)),
            pl.BlockSpec((bn, H), lambda n: (n, 0)),
        ],
        out_shape=[
            jax.ShapeDtypeStruct((N, H), jnp.float32),
            jax.ShapeDtypeStruct((N, H), jnp.float32),
        ],
    )(h, agg2, wu, bu.reshape(1, H), wnext)


def _final_body(h_ref, agg_ref, wu_ref, bu_ref, w1_ref, b1_ref, batch_ref,
                out_ref):
    agg = (agg_ref[0].astype(jnp.float32) + agg_ref[1].astype(jnp.float32))
    up = _ssp(
        jnp.dot(agg, wu_ref[...], preferred_element_type=jnp.float32) +
        bu_ref[...])
    h = h_ref[...] + up
    z = _ssp(
        jnp.dot(h, w1_ref[...], preferred_element_type=jnp.float32) +
        b1_ref[...])
    b = batch_ref[0, 0]
    onehot = (b[None, :] == lax.broadcasted_iota(jnp.int32, (NG, 1),
                                                 0)).astype(jnp.float32)
    part = jnp.dot(onehot, z, preferred_element_type=jnp.float32)

    @pl.when(pl.program_id(0) == 0)
    def _():
        out_ref[...] = jnp.zeros_like(out_ref)

    out_ref[...] += part


def _tc_final(h, agg2, wu, bu, w1, b1, batch, bn):
    grid = (N // bn,)
    return pl.pallas_call(
        _final_body,
        grid=grid,
        in_specs=[
            pl.BlockSpec((bn, H), lambda n: (n, 0)),
            pl.BlockSpec((2, bn, H), lambda n: (0, n, 0)),
            pl.BlockSpec((H, H), lambda n: (0, 0)),
            pl.BlockSpec((1, H), lambda n: (0, 0)),
            pl.BlockSpec((H, L_OUT), lambda n: (0, 0)),
            pl.BlockSpec((1, L_OUT), lambda n: (0, 0)),
            pl.BlockSpec((1, 1, bn), lambda n: (n, 0, 0)),
        ],
        out_specs=pl.BlockSpec((NG, L_OUT), lambda n: (0, 0)),
        out_shape=jax.ShapeDtypeStruct((NG, L_OUT), jnp.float32),
    )(h, agg2, wu, bu.reshape(1, H), w1, b1.reshape(1, L_OUT),
      batch.reshape(N // bn, 1, bn))


def kernel(x, pos, batch, edge_index_3rd, W0, b0, Wg1, bg1, Wg2, bg2, Wmsg,
           Wupd, bupd, W1, b1):
    pos_flat = jnp.pad(pos, ((0, 0), (0, 1))).reshape(4 * N)
    ei = jnp.pad(edge_index_3rd.astype(jnp.int32),
                 ((0, 0), (0, E_PAD - E)))
    idx_flat = ei.reshape(4 * E_PAD)
    off_d = jnp.linspace(0.0, CUTOFF, 50)
    off_a = jnp.linspace(0.0, math.pi, 6)
    off_t = jnp.linspace(-math.pi, math.pi, 12)
    coef_d = jnp.full((50,), -0.5 / (CUTOFF / 49.0) ** 2)
    coef_a = jnp.full((6,), -0.5 / (math.pi / 5.0) ** 2)
    coef_t = jnp.full((12,), -0.5 / (2.0 * math.pi / 11.0) ** 2)
    pad0 = jnp.zeros((128 - G_TOTAL,))
    offs = jnp.concatenate([off_d, off_a, off_t, pad0]).astype(
        jnp.float32).reshape(1, 128)
    coefs = jnp.concatenate([coef_d, coef_a, coef_t, pad0]).astype(
        jnp.float32).reshape(1, 128)
    wg1p = jnp.pad(Wg1, ((0, 0), (0, 128 - G_TOTAL), (0, 0)))

    p4 = _sc_gather_pos(pos_flat, idx_flat).reshape(3, 4, E_PAD)

    g_all = _tc_gates(p4, wg1p, bg1, Wg2, bg2, offs, coefs, be=2048)

    ii = ei[0]
    jj = ei[1]

    h, hm = _tc_h0(x, W0, b0, Wmsg[0], bn=2000)
    for t in range(3):
        agg2 = _sc_gather_mul_scatter(hm, g_all[t], jj, ii)
        if t < 2:
            h, hm = _tc_update(h, agg2, Wupd[t], bupd[t], Wmsg[t + 1],
                               bn=2000)
        else:
            out = _tc_final(h, agg2, Wupd[t], bupd[t], W1, b1, batch,
                            bn=2000)
    return out


# speedup vs baseline: 1.2374x; 1.2374x over previous
import functools
import math

import jax
import jax.numpy as jnp
from jax import lax
from jax.experimental import pallas as pl
from jax.experimental.pallas import tpu as pltpu
from jax.experimental.pallas import tpu_sc as plsc

N = 10000
N_PAD = 10240
E = 160000
E_PAD = 163840
F_IN = 5
H = 128
L_OUT = 64
NG = 64
CUTOFF = 10.0
G_TOTAL = 68
PD = 16

NCORES = 2
NSUB = 16
NW = NCORES * NSUB
CHUNK = 128


def _sc_gather_pos(pos_flat, idx):
    B = idx.shape[0]
    per_w = B // NW
    nchunk = per_w // CHUNK
    mesh = plsc.VectorSubcoreMesh(core_axis_name="c", subcore_axis_name="s")

    @functools.partial(
        pl.kernel,
        mesh=mesh,
        out_type=jax.ShapeDtypeStruct((3, B), jnp.float32),
        compiler_params=pltpu.CompilerParams(needs_layout_passes=False),
        scratch_types=[
            pltpu.VMEM((4 * N,), jnp.float32),
            pltpu.VMEM((CHUNK,), jnp.int32),
            pltpu.VMEM((3, CHUNK), jnp.float32),
            pltpu.SemaphoreType.DMA,
        ],
    )
    def k(tab_hbm, idx_hbm, out_hbm, tab_v, idx_v, out_v, sem):
        wid = lax.axis_index("s") * NCORES + lax.axis_index("c")
        base = wid * per_w
        pltpu.sync_copy(tab_hbm, tab_v)

        def body(c, _):
            off = base + c * CHUNK
            pltpu.sync_copy(idx_hbm.at[pl.ds(off, CHUNK)], idx_v)
            for s in range(CHUNK // 16):
                sl = pl.ds(s * 16, 16)
                addr = idx_v[sl] * 4
                for comp in range(3):
                    out_v[comp, sl] = plsc.load_gather(tab_v, [addr + comp])
            pltpu.sync_copy(out_v, out_hbm.at[:, pl.ds(off, CHUNK)])
            return ()

        lax.fori_loop(0, nchunk, body, ())

    return k(pos_flat, idx)


CH2 = 40


def _sc_gather_mul_scatter(hm, g, jj, ii):
    per_w = E_PAD // NW
    nchunk = per_w // CH2
    rows_per_tile = N_PAD // NSUB
    mesh = plsc.VectorSubcoreMesh(core_axis_name="c", subcore_axis_name="s")

    @functools.partial(
        pl.kernel,
        mesh=mesh,
        out_type=jax.ShapeDtypeStruct((NCORES, N_PAD, H), jnp.float32),
        compiler_params=pltpu.CompilerParams(needs_layout_passes=False),
        scratch_types=[
            pltpu.VMEM((2, CH2), jnp.int32),
            pltpu.VMEM((4, CH2), jnp.int32),
            pltpu.VMEM((2, CH2, H), jnp.float32),
            pltpu.VMEM((2, CH2, H), jnp.float32),
            pltpu.VMEM((2, CH2, H), jnp.float32),
            pltpu.VMEM_SHARED((N_PAD, H), jnp.float32),
        ] + [pltpu.SemaphoreType.DMA] * 12,
    )
    def k(hm_hbm, g_hbm, j_hbm, i_hbm, out_hbm, jv, iv, rows_v, g_v, msg_v,
          acc_sh, sj0, sj1, si0, si1, si2, si3, sg0, sg1, sr0, sr1, ss0,
          ss1):
        sj = (sj0, sj1)
        si = (si0, si1, si2, si3)
        sg = (sg0, sg1)
        sr = (sr0, sr1)
        ss = (ss0, ss1)
        cid = lax.axis_index("c")
        sid = lax.axis_index("s")
        wid = sid * NCORES + cid
        base = wid * per_w

        def zrow(r, _):
            for cc in range(H // 16):
                msg_v[0, r, pl.ds(cc * 16, 16)] = jnp.zeros(
                    (16,), jnp.float32)
            return ()

        lax.fori_loop(0, CH2, zrow, ())

        def zcopy(z, _):
            pltpu.sync_copy(
                msg_v.at[0],
                acc_sh.at[pl.ds(sid * rows_per_tile + z * CH2, CH2)])
            return ()

        lax.fori_loop(0, rows_per_tile // CH2, zcopy, ())
        plsc.subcore_barrier()

        def start_loads(c, b, b4):
            off = base + c * CH2
            pltpu.async_copy(j_hbm.at[pl.ds(off, CH2)], jv.at[b], sj[b])
            pltpu.async_copy(i_hbm.at[pl.ds(off, CH2)], iv.at[b4], si[b4])
            pltpu.async_copy(g_hbm.at[pl.ds(off, CH2)], g_v.at[b], sg[b])

        def start_gather(b):
            pltpu.async_copy(hm_hbm.at[jv.at[b]], rows_v.at[b], sr[b])

        def wait(sem, src, dst):
            pltpu.make_async_copy(src, dst, sem).wait()

        start_loads(0, 0, 0)
        start_loads(1, 1, 1)
        wait(sj[0], j_hbm.at[pl.ds(base, CH2)], jv.at[0])
        start_gather(0)

        def body(k4, _):
            for b4 in range(4):
                c = 4 * k4 + b4
                b = b4 % 2
                bn = 1 - b
                b4n = (b4 + 2) % 4

                @pl.when(c + 1 < nchunk)
                def _():
                    wait(sj[bn], j_hbm.at[pl.ds(base, CH2)], jv.at[bn])
                    start_gather(bn)

                @pl.when(c >= 2)
                def _():
                    wait(ss[b], msg_v.at[b], acc_sh.at[iv.at[b4]])

                wait(sr[b], hm_hbm.at[jv.at[b]], rows_v.at[b])
                wait(sg[b], g_hbm.at[pl.ds(base, CH2)], g_v.at[b])

                def mul_row(r, _):
                    for cc in range(H // 16):
                        sl = pl.ds(cc * 16, 16)
                        msg_v[b, r, sl] = rows_v[b, r, sl] * g_v[b, r, sl]
                    return ()

                lax.fori_loop(0, CH2, mul_row, ())

                wait(si[b4], i_hbm.at[pl.ds(base, CH2)], iv.at[b4])
                pltpu.async_copy(msg_v.at[b], acc_sh.at[iv.at[b4]], ss[b],
                                 add=True)

                @pl.when(c + 2 < nchunk)
                def _():
                    start_loads(c + 2, b, b4n)

            return ()

        lax.fori_loop(0, nchunk // 4, body, ())
        wait(ss[0], msg_v.at[0], acc_sh.at[iv.at[0]])
        wait(ss[1], msg_v.at[1], acc_sh.at[iv.at[1]])
        plsc.subcore_barrier()
        pltpu.sync_copy(
            acc_sh.at[pl.ds(sid * rows_per_tile, rows_per_tile)],
            out_hbm.at[cid, pl.ds(sid * rows_per_tile, rows_per_tile)])

    return k(hm, g, jj, ii)


def _ssp(v):
    return jax.nn.softplus(v) - math.log(2.0)


def _gate_body(p_ref, wg1_ref, bg1_ref, wg2_ref, bg2_ref, off_ref, coef_ref,
               g_ref, *, be):
    eps = 1e-8
    p = p_ref[...]

    def comp(a, c):
        return p[c, a, :]

    pix, piy, piz = comp(0, 0), comp(0, 1), comp(0, 2)
    pjx, pjy, pjz = comp(1, 0), comp(1, 1), comp(1, 2)
    pkx, pky, pkz = comp(2, 0), comp(2, 1), comp(2, 2)
    plx, ply, plz = comp(3, 0), comp(3, 1), comp(3, 2)

    b1x, b1y, b1z = pjx - pix, pjy - piy, pjz - piz
    b2x, b2y, b2z = pkx - pjx, pky - pjy, pkz - pjz
    b3x, b3y, b3z = plx - pkx, ply - pky, plz - pkz

    dist = jnp.sqrt(b1x * b1x + b1y * b1y + b1z * b1z + eps)

    dot12 = b1x * b2x + b1y * b2y + b1z * b2z
    n_v1 = jnp.sqrt(b1x * b1x + b1y * b1y + b1z * b1z)
    n_v2 = jnp.sqrt(b2x * b2x + b2y * b2y + b2z * b2z)
    cos_a = -dot12 / (n_v1 * n_v2 + eps)
    cos_a = jnp.clip(cos_a, -1.0 + 1e-7, 1.0 - 1e-7)
    ang = jnp.arctan2(jnp.sqrt(1.0 - cos_a * cos_a), cos_a)

    n1x = b1y * b2z - b1z * b2y
    n1y = b1z * b2x - b1x * b2z
    n1z = b1x * b2y - b1y * b2x
    n2x = b2y * b3z - b2z * b3y
    n2y = b2z * b3x - b2x * b3z
    n2z = b2x * b3y - b2y * b3x
    inv_nb2 = 1.0 / (jnp.sqrt(b2x * b2x + b2y * b2y + b2z * b2z) + eps)
    ux, uy, uz = b2x * inv_nb2, b2y * inv_nb2, b2z * inv_nb2
    m1x = n1y * uz - n1z * uy
    m1y = n1z * ux - n1x * uz
    m1z = n1x * uy - n1y * ux
    yv = m1x * n2x + m1y * n2y + m1z * n2z
    xv = n1x * n2x + n1y * n2y + n1z * n2z
    tor = jnp.arctan2(yv, xv + eps)

    off = off_ref[...]
    coef = coef_ref[...]
    col = lax.broadcasted_iota(jnp.int32, (1, 128), 1)
    val = jnp.where(col < 50, dist[:, None],
                    jnp.where(col < 56, ang[:, None], tor[:, None]))
    dlt = val - off
    rbf = jnp.exp(coef * dlt * dlt) * (col < G_TOTAL).astype(jnp.float32)

    cut = 0.5 * (jnp.cos(dist * (math.pi / CUTOFF)) + 1.0)
    cut = cut * (dist < CUTOFF).astype(jnp.float32)
    row = pl.program_id(0) * be + lax.broadcasted_iota(jnp.int32, (be,), 0)
    cut = cut * (row < E).astype(jnp.float32)

    w1 = wg1_ref[...]
    bb1 = bg1_ref[...]
    w2 = wg2_ref[...]
    bb2 = bg2_ref[...]
    gm = _ssp(jnp.dot(rbf, w1, preferred_element_type=jnp.float32) + bb1)
    gt = _ssp(jnp.dot(gm, w2, preferred_element_type=jnp.float32) + bb2)
    g_ref[...] = gt * cut[:, None]


def _tc_gates_t(p4, wg1p_t, bg1_t, wg2_t, bg2_t, offs, coefs, be):
    grid = (E_PAD // be,)
    return pl.pallas_call(
        functools.partial(_gate_body, be=be),
        grid=grid,
        in_specs=[
            pl.BlockSpec((3, 4, be), lambda e: (0, 0, e)),
            pl.BlockSpec((128, H), lambda e: (0, 0)),
            pl.BlockSpec((1, H), lambda e: (0, 0)),
            pl.BlockSpec((H, H), lambda e: (0, 0)),
            pl.BlockSpec((1, H), lambda e: (0, 0)),
            pl.BlockSpec((1, 128), lambda e: (0, 0)),
            pl.BlockSpec((1, 128), lambda e: (0, 0)),
        ],
        out_specs=pl.BlockSpec((be, H), lambda e: (e, 0)),
        out_shape=jax.ShapeDtypeStruct((E_PAD, H), jnp.float32),
    )(p4, wg1p_t, bg1_t.reshape(1, H), wg2_t, bg2_t.reshape(1, H), offs,
      coefs)


def _h0_body(x_ref, w0_ref, b0_ref, wm_ref, h_ref, hm_ref):
    h = jnp.dot(x_ref[...], w0_ref[...],
                preferred_element_type=jnp.float32) + b0_ref[...]
    h_ref[...] = h
    hm_ref[...] = jnp.dot(h, wm_ref[...], preferred_element_type=jnp.float32)


def _tc_h0(x, w0, b0, wm0, bn):
    grid = (N // bn,)
    return pl.pallas_call(
        _h0_body,
        grid=grid,
        in_specs=[
            pl.BlockSpec((bn, F_IN), lambda n: (n, 0)),
            pl.BlockSpec((F_IN, H), lambda n: (0, 0)),
            pl.BlockSpec((1, H), lambda n: (0, 0)),
            pl.BlockSpec((H, H), lambda n: (0, 0)),
        ],
        out_specs=[
            pl.BlockSpec((bn, H), lambda n: (n, 0)),
            pl.BlockSpec((bn, H), lambda n: (n, 0)),
        ],
        out_shape=[
            jax.ShapeDtypeStruct((N, H), jnp.float32),
            jax.ShapeDtypeStruct((N, H), jnp.float32),
        ],
    )(x, w0, b0.reshape(1, H), wm0)


def _upd_body(h_ref, agg_ref, wu_ref, bu_ref, wn_ref, h_out_ref, hm_out_ref):
    agg = (agg_ref[0].astype(jnp.float32) + agg_ref[1].astype(jnp.float32))
    up = _ssp(
        jnp.dot(agg, wu_ref[...], preferred_element_type=jnp.float32) +
        bu_ref[...])
    h = h_ref[...] + up
    h_out_ref[...] = h
    hm_out_ref[...] = jnp.dot(h, wn_ref[...],
                              preferred_element_type=jnp.float32)


def _tc_update(h, agg2, wu, bu, wnext, bn):
    grid = (N // bn,)
    return pl.pallas_call(
        _upd_body,
        grid=grid,
        in_specs=[
            pl.BlockSpec((bn, H), lambda n: (n, 0)),
            pl.BlockSpec((2, bn, H), lambda n: (0, n, 0)),
            pl.BlockSpec((H, H), lambda n: (0, 0)),
            pl.BlockSpec((1, H), lambda n: (0, 0)),
            pl.BlockSpec((H, H), lambda n: (0, 0)),
        ],
        out_specs=[
            pl.BlockSpec((bn, H), lambda n: (n, 0)),
            pl.BlockSpec((bn, H), lambda n: (n, 0)),
        ],
        out_shape=[
            jax.ShapeDtypeStruct((N, H), jnp.float32),
            jax.ShapeDtypeStruct((N, H), jnp.float32),
        ],
    )(h, agg2, wu, bu.reshape(1, H), wnext)


def _final_body(h_ref, agg_ref, wu_ref, bu_ref, w1_ref, b1_ref, batch_ref,
                out_ref):
    agg = (agg_ref[0].astype(jnp.float32) + agg_ref[1].astype(jnp.float32))
    up = _ssp(
        jnp.dot(agg, wu_ref[...], preferred_element_type=jnp.float32) +
        bu_ref[...])
    h = h_ref[...] + up
    z = _ssp(
        jnp.dot(h, w1_ref[...], preferred_element_type=jnp.float32) +
        b1_ref[...])
    b = batch_ref[0, 0]
    onehot = (b[None, :] == lax.broadcasted_iota(jnp.int32, (NG, 1),
                                                 0)).astype(jnp.float32)
    part = jnp.dot(onehot, z, preferred_element_type=jnp.float32)

    @pl.when(pl.program_id(0) == 0)
    def _():
        out_ref[...] = jnp.zeros_like(out_ref)

    out_ref[...] += part


def _tc_final(h, agg2, wu, bu, w1, b1, batch, bn):
    grid = (N // bn,)
    return pl.pallas_call(
        _final_body,
        grid=grid,
        in_specs=[
            pl.BlockSpec((bn, H), lambda n: (n, 0)),
            pl.BlockSpec((2, bn, H), lambda n: (0, n, 0)),
            pl.BlockSpec((H, H), lambda n: (0, 0)),
            pl.BlockSpec((1, H), lambda n: (0, 0)),
            pl.BlockSpec((H, L_OUT), lambda n: (0, 0)),
            pl.BlockSpec((1, L_OUT), lambda n: (0, 0)),
            pl.BlockSpec((1, 1, bn), lambda n: (n, 0, 0)),
        ],
        out_specs=pl.BlockSpec((NG, L_OUT), lambda n: (0, 0)),
        out_shape=jax.ShapeDtypeStruct((NG, L_OUT), jnp.float32),
    )(h, agg2, wu, bu.reshape(1, H), w1, b1.reshape(1, L_OUT),
      batch.reshape(N // bn, 1, bn))


def kernel(x, pos, batch, edge_index_3rd, W0, b0, Wg1, bg1, Wg2, bg2, Wmsg,
           Wupd, bupd, W1, b1):
    pos_flat = jnp.pad(pos, ((0, 0), (0, 1))).reshape(4 * N)
    ei = jnp.pad(edge_index_3rd.astype(jnp.int32),
                 ((0, 0), (0, E_PAD - E)))
    idx_flat = ei.reshape(4 * E_PAD)
    off_d = jnp.linspace(0.0, CUTOFF, 50)
    off_a = jnp.linspace(0.0, math.pi, 6)
    off_t = jnp.linspace(-math.pi, math.pi, 12)
    coef_d = jnp.full((50,), -0.5 / (CUTOFF / 49.0) ** 2)
    coef_a = jnp.full((6,), -0.5 / (math.pi / 5.0) ** 2)
    coef_t = jnp.full((12,), -0.5 / (2.0 * math.pi / 11.0) ** 2)
    pad0 = jnp.zeros((128 - G_TOTAL,))
    offs = jnp.concatenate([off_d, off_a, off_t, pad0]).astype(
        jnp.float32).reshape(1, 128)
    coefs = jnp.concatenate([coef_d, coef_a, coef_t, pad0]).astype(
        jnp.float32).reshape(1, 128)
    wg1p = jnp.pad(Wg1, ((0, 0), (0, 128 - G_TOTAL), (0, 0)))

    p4 = _sc_gather_pos(pos_flat, idx_flat).reshape(3, 4, E_PAD)

    ii = ei[0]
    jj = ei[1]

    h, hm = _tc_h0(x, W0, b0, Wmsg[0], bn=2000)
    for t in range(3):
        g_t = _tc_gates_t(p4, wg1p[t], bg1[t], Wg2[t], bg2[t], offs, coefs,
                          be=2048)
        agg2 = _sc_gather_mul_scatter(hm, g_t, jj, ii)
        if t < 2:
            h, hm = _tc_update(h, agg2, Wupd[t], bupd[t], Wmsg[t + 1],
                               bn=2000)
        else:
            out = _tc_final(h, agg2, Wupd[t], bupd[t], W1, b1, batch,
                            bn=2000)
    return out


# speedup vs baseline: 1.3103x; 1.0590x over previous
import functools
import math

import jax
import jax.numpy as jnp
from jax import lax
from jax.experimental import pallas as pl
from jax.experimental.pallas import tpu as pltpu
from jax.experimental.pallas import tpu_sc as plsc

N = 10000
N_PAD = 10240
E = 160000
E_PAD = 163840
F_IN = 5
H = 128
L_OUT = 64
NG = 64
CUTOFF = 10.0
G_TOTAL = 68
PD = 16

NCORES = 2
NSUB = 16
NW = NCORES * NSUB
CHUNK = 128


def _sc_gather_pos(pos_flat, idx):
    B = idx.shape[0]
    per_w = B // NW
    nchunk = per_w // CHUNK
    mesh = plsc.VectorSubcoreMesh(core_axis_name="c", subcore_axis_name="s")

    @functools.partial(
        pl.kernel,
        mesh=mesh,
        out_type=jax.ShapeDtypeStruct((3, B), jnp.float32),
        compiler_params=pltpu.CompilerParams(needs_layout_passes=False),
        scratch_types=[
            pltpu.VMEM((4 * N,), jnp.float32),
            pltpu.VMEM((CHUNK,), jnp.int32),
            pltpu.VMEM((3, CHUNK), jnp.float32),
            pltpu.SemaphoreType.DMA,
        ],
    )
    def k(tab_hbm, idx_hbm, out_hbm, tab_v, idx_v, out_v, sem):
        wid = lax.axis_index("s") * NCORES + lax.axis_index("c")
        base = wid * per_w
        pltpu.sync_copy(tab_hbm, tab_v)

        def body(c, _):
            off = base + c * CHUNK
            pltpu.sync_copy(idx_hbm.at[pl.ds(off, CHUNK)], idx_v)
            for s in range(CHUNK // 16):
                sl = pl.ds(s * 16, 16)
                addr = idx_v[sl] * 4
                for comp in range(3):
                    out_v[comp, sl] = plsc.load_gather(tab_v, [addr + comp])
            pltpu.sync_copy(out_v, out_hbm.at[:, pl.ds(off, CHUNK)])
            return ()

        lax.fori_loop(0, nchunk, body, ())

    return k(pos_flat, idx)


CH2 = 40
NCF = 168
NCS = 88


def _sc_gather_mul_scatter(hm, g, jj, ii):
    rows_per_tile = N_PAD // NSUB
    mesh = plsc.VectorSubcoreMesh(core_axis_name="c", subcore_axis_name="s")

    @functools.partial(
        pl.kernel,
        mesh=mesh,
        out_type=jax.ShapeDtypeStruct((NCORES, N_PAD, H), jnp.float32),
        compiler_params=pltpu.CompilerParams(needs_layout_passes=False),
        scratch_types=[
            pltpu.VMEM((2, CH2), jnp.int32),
            pltpu.VMEM((4, CH2), jnp.int32),
            pltpu.VMEM((2, CH2, H), jnp.float32),
            pltpu.VMEM((2, CH2, H), jnp.float32),
            pltpu.VMEM((2, CH2, H), jnp.float32),
            pltpu.VMEM_SHARED((N_PAD, H), jnp.float32),
        ] + [pltpu.SemaphoreType.DMA] * 12,
    )
    def k(hm_hbm, g_hbm, j_hbm, i_hbm, out_hbm, jv, iv, rows_v, g_v, msg_v,
          acc_sh, sj0, sj1, si0, si1, si2, si3, sg0, sg1, sr0, sr1, ss0,
          ss1):
        sj = (sj0, sj1)
        si = (si0, si1, si2, si3)
        sg = (sg0, sg1)
        sr = (sr0, sr1)
        ss = (ss0, ss1)
        cid = lax.axis_index("c")
        sid = lax.axis_index("s")
        nchunk = jnp.where(cid == 0, NCF, NCS)
        base = jnp.where(cid == 0, sid * (NCF * CH2),
                         NSUB * NCF * CH2 + sid * (NCS * CH2))

        def zrow(r, _):
            for cc in range(H // 16):
                msg_v[0, r, pl.ds(cc * 16, 16)] = jnp.zeros(
                    (16,), jnp.float32)
            return ()

        lax.fori_loop(0, CH2, zrow, ())

        def zcopy(z, _):
            pltpu.sync_copy(
                msg_v.at[0],
                acc_sh.at[pl.ds(sid * rows_per_tile + z * CH2, CH2)])
            return ()

        lax.fori_loop(0, rows_per_tile // CH2, zcopy, ())
        plsc.subcore_barrier()

        def start_loads(c, b, b4):
            off = base + c * CH2
            pltpu.async_copy(j_hbm.at[pl.ds(off, CH2)], jv.at[b], sj[b])
            pltpu.async_copy(i_hbm.at[pl.ds(off, CH2)], iv.at[b4], si[b4])
            pltpu.async_copy(g_hbm.at[pl.ds(off, CH2)], g_v.at[b], sg[b])

        def start_gather(b):
            pltpu.async_copy(hm_hbm.at[jv.at[b]], rows_v.at[b], sr[b])

        def wait(sem, src, dst):
            pltpu.make_async_copy(src, dst, sem).wait()

        start_loads(0, 0, 0)
        start_loads(1, 1, 1)
        wait(sj[0], j_hbm.at[pl.ds(base, CH2)], jv.at[0])
        start_gather(0)

        def body(k4, _):
            for b4 in range(4):
                c = 4 * k4 + b4
                b = b4 % 2
                bn = 1 - b
                b4n = (b4 + 2) % 4

                @pl.when(c + 1 < nchunk)
                def _():
                    wait(sj[bn], j_hbm.at[pl.ds(base, CH2)], jv.at[bn])
                    start_gather(bn)

                @pl.when(c >= 2)
                def _():
                    wait(ss[b], msg_v.at[b], acc_sh.at[iv.at[b4]])

                wait(sr[b], hm_hbm.at[jv.at[b]], rows_v.at[b])
                wait(sg[b], g_hbm.at[pl.ds(base, CH2)], g_v.at[b])

                def mul_row(r, _):
                    for cc in range(H // 16):
                        sl = pl.ds(cc * 16, 16)
                        msg_v[b, r, sl] = rows_v[b, r, sl] * g_v[b, r, sl]
                    return ()

                lax.fori_loop(0, CH2, mul_row, ())

                wait(si[b4], i_hbm.at[pl.ds(base, CH2)], iv.at[b4])
                pltpu.async_copy(msg_v.at[b], acc_sh.at[iv.at[b4]], ss[b],
                                 add=True)

                @pl.when(c + 2 < nchunk)
                def _():
                    start_loads(c + 2, b, b4n)

            return ()

        lax.fori_loop(0, nchunk // 4, body, ())
        wait(ss[0], msg_v.at[0], acc_sh.at[iv.at[0]])
        wait(ss[1], msg_v.at[1], acc_sh.at[iv.at[1]])
        plsc.subcore_barrier()
        pltpu.sync_copy(
            acc_sh.at[pl.ds(sid * rows_per_tile, rows_per_tile)],
            out_hbm.at[cid, pl.ds(sid * rows_per_tile, rows_per_tile)])

    return k(hm, g, jj, ii)


def _ssp(v):
    return jax.nn.softplus(v) - math.log(2.0)


def _gate_body(p_ref, wg1_ref, bg1_ref, wg2_ref, bg2_ref, off_ref, coef_ref,
               g_ref, *, be):
    eps = 1e-8
    p = p_ref[...]

    def comp(a, c):
        return p[c, a, :]

    pix, piy, piz = comp(0, 0), comp(0, 1), comp(0, 2)
    pjx, pjy, pjz = comp(1, 0), comp(1, 1), comp(1, 2)
    pkx, pky, pkz = comp(2, 0), comp(2, 1), comp(2, 2)
    plx, ply, plz = comp(3, 0), comp(3, 1), comp(3, 2)

    b1x, b1y, b1z = pjx - pix, pjy - piy, pjz - piz
    b2x, b2y, b2z = pkx - pjx, pky - pjy, pkz - pjz
    b3x, b3y, b3z = plx - pkx, ply - pky, plz - pkz

    dist = jnp.sqrt(b1x * b1x + b1y * b1y + b1z * b1z + eps)

    dot12 = b1x * b2x + b1y * b2y + b1z * b2z
    n_v1 = jnp.sqrt(b1x * b1x + b1y * b1y + b1z * b1z)
    n_v2 = jnp.sqrt(b2x * b2x + b2y * b2y + b2z * b2z)
    cos_a = -dot12 / (n_v1 * n_v2 + eps)
    cos_a = jnp.clip(cos_a, -1.0 + 1e-7, 1.0 - 1e-7)
    ang = jnp.arctan2(jnp.sqrt(1.0 - cos_a * cos_a), cos_a)

    n1x = b1y * b2z - b1z * b2y
    n1y = b1z * b2x - b1x * b2z
    n1z = b1x * b2y - b1y * b2x
    n2x = b2y * b3z - b2z * b3y
    n2y = b2z * b3x - b2x * b3z
    n2z = b2x * b3y - b2y * b3x
    inv_nb2 = 1.0 / (jnp.sqrt(b2x * b2x + b2y * b2y + b2z * b2z) + eps)
    ux, uy, uz = b2x * inv_nb2, b2y * inv_nb2, b2z * inv_nb2
    m1x = n1y * uz - n1z * uy
    m1y = n1z * ux - n1x * uz
    m1z = n1x * uy - n1y * ux
    yv = m1x * n2x + m1y * n2y + m1z * n2z
    xv = n1x * n2x + n1y * n2y + n1z * n2z
    tor = jnp.arctan2(yv, xv + eps)

    off = off_ref[...]
    coef = coef_ref[...]
    col = lax.broadcasted_iota(jnp.int32, (1, 128), 1)
    val = jnp.where(col < 50, dist[:, None],
                    jnp.where(col < 56, ang[:, None], tor[:, None]))
    dlt = val - off
    rbf = jnp.exp(coef * dlt * dlt) * (col < G_TOTAL).astype(jnp.float32)

    cut = 0.5 * (jnp.cos(dist * (math.pi / CUTOFF)) + 1.0)
    cut = cut * (dist < CUTOFF).astype(jnp.float32)
    row = pl.program_id(0) * be + lax.broadcasted_iota(jnp.int32, (be,), 0)
    cut = cut * (row < E).astype(jnp.float32)

    w1 = wg1_ref[...]
    bb1 = bg1_ref[...]
    w2 = wg2_ref[...]
    bb2 = bg2_ref[...]
    gm = _ssp(jnp.dot(rbf, w1, preferred_element_type=jnp.float32) + bb1)
    gt = _ssp(jnp.dot(gm, w2, preferred_element_type=jnp.float32) + bb2)
    g_ref[...] = gt * cut[:, None]


def _tc_gates_t(p4, wg1p_t, bg1_t, wg2_t, bg2_t, offs, coefs, be):
    grid = (E_PAD // be,)
    return pl.pallas_call(
        functools.partial(_gate_body, be=be),
        grid=grid,
        in_specs=[
            pl.BlockSpec((3, 4, be), lambda e: (0, 0, e)),
            pl.BlockSpec((128, H), lambda e: (0, 0)),
            pl.BlockSpec((1, H), lambda e: (0, 0)),
            pl.BlockSpec((H, H), lambda e: (0, 0)),
            pl.BlockSpec((1, H), lambda e: (0, 0)),
            pl.BlockSpec((1, 128), lambda e: (0, 0)),
            pl.BlockSpec((1, 128), lambda e: (0, 0)),
        ],
        out_specs=pl.BlockSpec((be, H), lambda e: (e, 0)),
        out_shape=jax.ShapeDtypeStruct((E_PAD, H), jnp.float32),
    )(p4, wg1p_t, bg1_t.reshape(1, H), wg2_t, bg2_t.reshape(1, H), offs,
      coefs)


def _h0_body(x_ref, w0_ref, b0_ref, wm_ref, h_ref, hm_ref):
    h = jnp.dot(x_ref[...], w0_ref[...],
                preferred_element_type=jnp.float32) + b0_ref[...]
    h_ref[...] = h
    hm_ref[...] = jnp.dot(h, wm_ref[...], preferred_element_type=jnp.float32)


def _tc_h0(x, w0, b0, wm0, bn):
    grid = (N // bn,)
    return pl.pallas_call(
        _h0_body,
        grid=grid,
        in_specs=[
            pl.BlockSpec((bn, F_IN), lambda n: (n, 0)),
            pl.BlockSpec((F_IN, H), lambda n: (0, 0)),
            pl.BlockSpec((1, H), lambda n: (0, 0)),
            pl.BlockSpec((H, H), lambda n: (0, 0)),
        ],
        out_specs=[
            pl.BlockSpec((bn, H), lambda n: (n, 0)),
            pl.BlockSpec((bn, H), lambda n: (n, 0)),
        ],
        out_shape=[
            jax.ShapeDtypeStruct((N, H), jnp.float32),
            jax.ShapeDtypeStruct((N, H), jnp.float32),
        ],
    )(x, w0, b0.reshape(1, H), wm0)


def _upd_body(h_ref, agg_ref, wu_ref, bu_ref, wn_ref, h_out_ref, hm_out_ref):
    agg = (agg_ref[0].astype(jnp.float32) + agg_ref[1].astype(jnp.float32))
    up = _ssp(
        jnp.dot(agg, wu_ref[...], preferred_element_type=jnp.float32) +
        bu_ref[...])
    h = h_ref[...] + up
    h_out_ref[...] = h
    hm_out_ref[...] = jnp.dot(h, wn_ref[...],
                              preferred_element_type=jnp.float32)


def _tc_update(h, agg2, wu, bu, wnext, bn):
    grid = (N // bn,)
    return pl.pallas_call(
        _upd_body,
        grid=grid,
        in_specs=[
            pl.BlockSpec((bn, H), lambda n: (n, 0)),
            pl.BlockSpec((2, bn, H), lambda n: (0, n, 0)),
            pl.BlockSpec((H, H), lambda n: (0, 0)),
            pl.BlockSpec((1, H), lambda n: (0, 0)),
            pl.BlockSpec((H, H), lambda n: (0, 0)),
        ],
        out_specs=[
            pl.BlockSpec((bn, H), lambda n: (n, 0)),
            pl.BlockSpec((bn, H), lambda n: (n, 0)),
        ],
        out_shape=[
            jax.ShapeDtypeStruct((N, H), jnp.float32),
            jax.ShapeDtypeStruct((N, H), jnp.float32),
        ],
    )(h, agg2, wu, bu.reshape(1, H), wnext)


def _final_body(h_ref, agg_ref, wu_ref, bu_ref, w1_ref, b1_ref, batch_ref,
                out_ref):
    agg = (agg_ref[0].astype(jnp.float32) + agg_ref[1].astype(jnp.float32))
    up = _ssp(
        jnp.dot(agg, wu_ref[...], preferred_element_type=jnp.float32) +
        bu_ref[...])
    h = h_ref[...] + up
    z = _ssp(
        jnp.dot(h, w1_ref[...], preferred_element_type=jnp.float32) +
        b1_ref[...])
    b = batch_ref[0, 0]
    onehot = (b[None, :] == lax.broadcasted_iota(jnp.int32, (NG, 1),
                                                 0)).astype(jnp.float32)
    part = jnp.dot(onehot, z, preferred_element_type=jnp.float32)

    @pl.when(pl.program_id(0) == 0)
    def _():
        out_ref[...] = jnp.zeros_like(out_ref)

    out_ref[...] += part


def _tc_final(h, agg2, wu, bu, w1, b1, batch, bn):
    grid = (N // bn,)
    return pl.pallas_call(
        _final_body,
        grid=grid,
        in_specs=[
            pl.BlockSpec((bn, H), lambda n: (n, 0)),
            pl.BlockSpec((2, bn, H), lambda n: (0, n, 0)),
            pl.BlockSpec((H, H), lambda n: (0, 0)),
            pl.BlockSpec((1, H), lambda n: (0, 0)),
            pl.BlockSpec((H, L_OUT), lambda n: (0, 0)),
            pl.BlockSpec((1, L_OUT), lambda n: (0, 0)),
            pl.BlockSpec((1, 1, bn), lambda n: (n, 0, 0)),
        ],
        out_specs=pl.BlockSpec((NG, L_OUT), lambda n: (0, 0)),
        out_shape=jax.ShapeDtypeStruct((NG, L_OUT), jnp.float32),
    )(h, agg2, wu, bu.reshape(1, H), w1, b1.reshape(1, L_OUT),
      batch.reshape(N // bn, 1, bn))


def kernel(x, pos, batch, edge_index_3rd, W0, b0, Wg1, bg1, Wg2, bg2, Wmsg,
           Wupd, bupd, W1, b1):
    pos_flat = jnp.pad(pos, ((0, 0), (0, 1))).reshape(4 * N)
    ei = jnp.pad(edge_index_3rd.astype(jnp.int32),
                 ((0, 0), (0, E_PAD - E)))
    idx_flat = ei.reshape(4 * E_PAD)
    off_d = jnp.linspace(0.0, CUTOFF, 50)
    off_a = jnp.linspace(0.0, math.pi, 6)
    off_t = jnp.linspace(-math.pi, math.pi, 12)
    coef_d = jnp.full((50,), -0.5 / (CUTOFF / 49.0) ** 2)
    coef_a = jnp.full((6,), -0.5 / (math.pi / 5.0) ** 2)
    coef_t = jnp.full((12,), -0.5 / (2.0 * math.pi / 11.0) ** 2)
    pad0 = jnp.zeros((128 - G_TOTAL,))
    offs = jnp.concatenate([off_d, off_a, off_t, pad0]).astype(
        jnp.float32).reshape(1, 128)
    coefs = jnp.concatenate([coef_d, coef_a, coef_t, pad0]).astype(
        jnp.float32).reshape(1, 128)
    wg1p = jnp.pad(Wg1, ((0, 0), (0, 128 - G_TOTAL), (0, 0)))

    p4 = _sc_gather_pos(pos_flat, idx_flat).reshape(3, 4, E_PAD)

    ii = ei[0]
    jj = ei[1]

    h, hm = _tc_h0(x, W0, b0, Wmsg[0], bn=2000)
    for t in range(3):
        g_t = _tc_gates_t(p4, wg1p[t], bg1[t], Wg2[t], bg2[t], offs, coefs,
                          be=2048)
        agg2 = _sc_gather_mul_scatter(hm, g_t, jj, ii)
        if t < 2:
            h, hm = _tc_update(h, agg2, Wupd[t], bupd[t], Wmsg[t + 1],
                               bn=2000)
        else:
            out = _tc_final(h, agg2, Wupd[t], bupd[t], W1, b1, batch,
                            bn=2000)
    return out


# speedup vs baseline: 1.4087x; 1.0751x over previous
import functools
import math

import jax
import jax.numpy as jnp
from jax import lax
from jax.experimental import pallas as pl
from jax.experimental.pallas import tpu as pltpu
from jax.experimental.pallas import tpu_sc as plsc

N = 10000
N_PAD = 10240
E = 160000
E_PAD = 163840
F_IN = 5
H = 128
L_OUT = 64
NG = 64
CUTOFF = 10.0
G_TOTAL = 68
PD = 16

NCORES = 2
NSUB = 16
NW = NCORES * NSUB
CHUNK = 128


def _sc_gather_pos(pos_flat, idx):
    B = idx.shape[0]
    per_w = B // NW
    CH1 = 512
    nchunk = per_w // CH1
    mesh = plsc.VectorSubcoreMesh(core_axis_name="c", subcore_axis_name="s")

    @functools.partial(
        pl.kernel,
        mesh=mesh,
        out_type=jax.ShapeDtypeStruct((3, B), jnp.float32),
        compiler_params=pltpu.CompilerParams(needs_layout_passes=False),
        scratch_types=[
            pltpu.VMEM((4 * N,), jnp.float32),
            pltpu.VMEM((2, CH1), jnp.int32),
            pltpu.VMEM((2, 3, CH1), jnp.float32),
        ] + [pltpu.SemaphoreType.DMA] * 4,
    )
    def k(tab_hbm, idx_hbm, out_hbm, tab_v, idx_v, out_v, sj0, sj1, so0,
          so1):
        sj = (sj0, sj1)
        so = (so0, so1)
        wid = lax.axis_index("s") * NCORES + lax.axis_index("c")
        base = wid * per_w
        pltpu.sync_copy(tab_hbm, tab_v)

        def wait(sem, src, dst):
            pltpu.make_async_copy(src, dst, sem).wait()

        pltpu.async_copy(idx_hbm.at[pl.ds(base, CH1)], idx_v.at[0], sj[0])

        def body(k2, _):
            for b in (0, 1):
                c = 2 * k2 + b
                bn = 1 - b
                off = base + c * CH1

                @pl.when(c + 1 < nchunk)
                def _():
                    pltpu.async_copy(
                        idx_hbm.at[pl.ds(off + CH1, CH1)], idx_v.at[bn],
                        sj[bn])

                @pl.when(c >= 2)
                def _():
                    wait(so[b], out_v.at[b],
                         out_hbm.at[:, pl.ds(base, CH1)])

                wait(sj[b], idx_hbm.at[pl.ds(base, CH1)], idx_v.at[b])
                for s in range(CH1 // 16):
                    sl = pl.ds(s * 16, 16)
                    addr = idx_v[b, sl] * 4
                    for comp in range(3):
                        out_v[b, comp, sl] = plsc.load_gather(
                            tab_v, [addr + comp])
                pltpu.async_copy(out_v.at[b],
                                 out_hbm.at[:, pl.ds(off, CH1)], so[b])
            return ()

        lax.fori_loop(0, nchunk // 2, body, ())
        wait(so[0], out_v.at[0], out_hbm.at[:, pl.ds(base, CH1)])
        wait(so[1], out_v.at[1], out_hbm.at[:, pl.ds(base, CH1)])

    return k(pos_flat, idx)


CH2 = 40
NCF = 168
NCS = 88


def _sc_gather_mul_scatter(hm, g, jj, ii):
    rows_per_tile = N_PAD // NSUB
    mesh = plsc.VectorSubcoreMesh(core_axis_name="c", subcore_axis_name="s")

    @functools.partial(
        pl.kernel,
        mesh=mesh,
        out_type=jax.ShapeDtypeStruct((NCORES, N_PAD, H), jnp.float32),
        compiler_params=pltpu.CompilerParams(needs_layout_passes=False),
        scratch_types=[
            pltpu.VMEM((2, CH2), jnp.int32),
            pltpu.VMEM((4, CH2), jnp.int32),
            pltpu.VMEM((2, CH2, H), jnp.float32),
            pltpu.VMEM((2, CH2, H), jnp.float32),
            pltpu.VMEM((2, CH2, H), jnp.float32),
            pltpu.VMEM_SHARED((N_PAD, H), jnp.float32),
        ] + [pltpu.SemaphoreType.DMA] * 12,
    )
    def k(hm_hbm, g_hbm, j_hbm, i_hbm, out_hbm, jv, iv, rows_v, g_v, msg_v,
          acc_sh, sj0, sj1, si0, si1, si2, si3, sg0, sg1, sr0, sr1, ss0,
          ss1):
        sj = (sj0, sj1)
        si = (si0, si1, si2, si3)
        sg = (sg0, sg1)
        sr = (sr0, sr1)
        ss = (ss0, ss1)
        cid = lax.axis_index("c")
        sid = lax.axis_index("s")
        nchunk = jnp.where(cid == 0, NCF, NCS)
        base = jnp.where(cid == 0, sid * (NCF * CH2),
                         NSUB * NCF * CH2 + sid * (NCS * CH2))

        def zrow(r, _):
            for cc in range(H // 16):
                msg_v[0, r, pl.ds(cc * 16, 16)] = jnp.zeros(
                    (16,), jnp.float32)
            return ()

        lax.fori_loop(0, CH2, zrow, ())

        def zcopy(z, _):
            pltpu.sync_copy(
                msg_v.at[0],
                acc_sh.at[pl.ds(sid * rows_per_tile + z * CH2, CH2)])
            return ()

        lax.fori_loop(0, rows_per_tile // CH2, zcopy, ())
        plsc.subcore_barrier()

        def start_loads(c, b, b4):
            off = base + c * CH2
            pltpu.async_copy(j_hbm.at[pl.ds(off, CH2)], jv.at[b], sj[b])
            pltpu.async_copy(i_hbm.at[pl.ds(off, CH2)], iv.at[b4], si[b4])
            pltpu.async_copy(g_hbm.at[pl.ds(off, CH2)], g_v.at[b], sg[b])

        def start_gather(b):
            pltpu.async_copy(hm_hbm.at[jv.at[b]], rows_v.at[b], sr[b])

        def wait(sem, src, dst):
            pltpu.make_async_copy(src, dst, sem).wait()

        start_loads(0, 0, 0)
        start_loads(1, 1, 1)
        wait(sj[0], j_hbm.at[pl.ds(base, CH2)], jv.at[0])
        start_gather(0)

        def body(k4, _):
            for b4 in range(4):
                c = 4 * k4 + b4
                b = b4 % 2
                bn = 1 - b
                b4n = (b4 + 2) % 4

                @pl.when(c + 1 < nchunk)
                def _():
                    wait(sj[bn], j_hbm.at[pl.ds(base, CH2)], jv.at[bn])
                    start_gather(bn)

                @pl.when(c >= 2)
                def _():
                    wait(ss[b], msg_v.at[b], acc_sh.at[iv.at[b4]])

                wait(sr[b], hm_hbm.at[jv.at[b]], rows_v.at[b])
                wait(sg[b], g_hbm.at[pl.ds(base, CH2)], g_v.at[b])

                def mul_row(r, _):
                    for cc in range(H // 16):
                        sl = pl.ds(cc * 16, 16)
                        msg_v[b, r, sl] = rows_v[b, r, sl] * g_v[b, r, sl]
                    return ()

                lax.fori_loop(0, CH2, mul_row, ())

                wait(si[b4], i_hbm.at[pl.ds(base, CH2)], iv.at[b4])
                pltpu.async_copy(msg_v.at[b], acc_sh.at[iv.at[b4]], ss[b],
                                 add=True)

                @pl.when(c + 2 < nchunk)
                def _():
                    start_loads(c + 2, b, b4n)

            return ()

        lax.fori_loop(0, nchunk // 4, body, ())
        wait(ss[0], msg_v.at[0], acc_sh.at[iv.at[0]])
        wait(ss[1], msg_v.at[1], acc_sh.at[iv.at[1]])
        plsc.subcore_barrier()
        pltpu.sync_copy(
            acc_sh.at[pl.ds(sid * rows_per_tile, rows_per_tile)],
            out_hbm.at[cid, pl.ds(sid * rows_per_tile, rows_per_tile)])

    return k(hm, g, jj, ii)


def _ssp(v):
    return jax.nn.softplus(v) - math.log(2.0)


def _gate_body(p_ref, wg1_ref, bg1_ref, wg2_ref, bg2_ref, off_ref, coef_ref,
               g_ref, *, be):
    eps = 1e-8
    p = p_ref[...]

    def comp(a, c):
        return p[c, a, :]

    pix, piy, piz = comp(0, 0), comp(0, 1), comp(0, 2)
    pjx, pjy, pjz = comp(1, 0), comp(1, 1), comp(1, 2)
    pkx, pky, pkz = comp(2, 0), comp(2, 1), comp(2, 2)
    plx, ply, plz = comp(3, 0), comp(3, 1), comp(3, 2)

    b1x, b1y, b1z = pjx - pix, pjy - piy, pjz - piz
    b2x, b2y, b2z = pkx - pjx, pky - pjy, pkz - pjz
    b3x, b3y, b3z = plx - pkx, ply - pky, plz - pkz

    dist = jnp.sqrt(b1x * b1x + b1y * b1y + b1z * b1z + eps)

    dot12 = b1x * b2x + b1y * b2y + b1z * b2z
    n_v1 = jnp.sqrt(b1x * b1x + b1y * b1y + b1z * b1z)
    n_v2 = jnp.sqrt(b2x * b2x + b2y * b2y + b2z * b2z)
    cos_a = -dot12 / (n_v1 * n_v2 + eps)
    cos_a = jnp.clip(cos_a, -1.0 + 1e-7, 1.0 - 1e-7)
    ang = jnp.arctan2(jnp.sqrt(1.0 - cos_a * cos_a), cos_a)

    n1x = b1y * b2z - b1z * b2y
    n1y = b1z * b2x - b1x * b2z
    n1z = b1x * b2y - b1y * b2x
    n2x = b2y * b3z - b2z * b3y
    n2y = b2z * b3x - b2x * b3z
    n2z = b2x * b3y - b2y * b3x
    inv_nb2 = 1.0 / (jnp.sqrt(b2x * b2x + b2y * b2y + b2z * b2z) + eps)
    ux, uy, uz = b2x * inv_nb2, b2y * inv_nb2, b2z * inv_nb2
    m1x = n1y * uz - n1z * uy
    m1y = n1z * ux - n1x * uz
    m1z = n1x * uy - n1y * ux
    yv = m1x * n2x + m1y * n2y + m1z * n2z
    xv = n1x * n2x + n1y * n2y + n1z * n2z
    tor = jnp.arctan2(yv, xv + eps)

    off = off_ref[...]
    coef = coef_ref[...]
    col = lax.broadcasted_iota(jnp.int32, (1, 128), 1)
    val = jnp.where(col < 50, dist[:, None],
                    jnp.where(col < 56, ang[:, None], tor[:, None]))
    dlt = val - off
    rbf = jnp.exp(coef * dlt * dlt) * (col < G_TOTAL).astype(jnp.float32)

    cut = 0.5 * (jnp.cos(dist * (math.pi / CUTOFF)) + 1.0)
    cut = cut * (dist < CUTOFF).astype(jnp.float32)
    row = pl.program_id(0) * be + lax.broadcasted_iota(jnp.int32, (be,), 0)
    cut = cut * (row < E).astype(jnp.float32)

    w1 = wg1_ref[...]
    bb1 = bg1_ref[...]
    w2 = wg2_ref[...]
    bb2 = bg2_ref[...]
    gm = _ssp(jnp.dot(rbf, w1, preferred_element_type=jnp.float32) + bb1)
    gt = _ssp(jnp.dot(gm, w2, preferred_element_type=jnp.float32) + bb2)
    g_ref[...] = gt * cut[:, None]


def _tc_gates_t(p4, wg1p_t, bg1_t, wg2_t, bg2_t, offs, coefs, be):
    grid = (E_PAD // be,)
    return pl.pallas_call(
        functools.partial(_gate_body, be=be),
        grid=grid,
        in_specs=[
            pl.BlockSpec((3, 4, be), lambda e: (0, 0, e)),
            pl.BlockSpec((128, H), lambda e: (0, 0)),
            pl.BlockSpec((1, H), lambda e: (0, 0)),
            pl.BlockSpec((H, H), lambda e: (0, 0)),
            pl.BlockSpec((1, H), lambda e: (0, 0)),
            pl.BlockSpec((1, 128), lambda e: (0, 0)),
            pl.BlockSpec((1, 128), lambda e: (0, 0)),
        ],
        out_specs=pl.BlockSpec((be, H), lambda e: (e, 0)),
        out_shape=jax.ShapeDtypeStruct((E_PAD, H), jnp.float32),
    )(p4, wg1p_t, bg1_t.reshape(1, H), wg2_t, bg2_t.reshape(1, H), offs,
      coefs)


def _h0_body(x_ref, w0_ref, b0_ref, wm_ref, h_ref, hm_ref):
    h = jnp.dot(x_ref[...], w0_ref[...],
                preferred_element_type=jnp.float32) + b0_ref[...]
    h_ref[...] = h
    hm_ref[...] = jnp.dot(h, wm_ref[...], preferred_element_type=jnp.float32)


def _tc_h0(x, w0, b0, wm0, bn):
    grid = (N // bn,)
    return pl.pallas_call(
        _h0_body,
        grid=grid,
        in_specs=[
            pl.BlockSpec((bn, F_IN), lambda n: (n, 0)),
            pl.BlockSpec((F_IN, H), lambda n: (0, 0)),
            pl.BlockSpec((1, H), lambda n: (0, 0)),
            pl.BlockSpec((H, H), lambda n: (0, 0)),
        ],
        out_specs=[
            pl.BlockSpec((bn, H), lambda n: (n, 0)),
            pl.BlockSpec((bn, H), lambda n: (n, 0)),
        ],
        out_shape=[
            jax.ShapeDtypeStruct((N, H), jnp.float32),
            jax.ShapeDtypeStruct((N, H), jnp.float32),
        ],
    )(x, w0, b0.reshape(1, H), wm0)


def _upd_body(h_ref, agg_ref, wu_ref, bu_ref, wn_ref, h_out_ref, hm_out_ref):
    agg = (agg_ref[0].astype(jnp.float32) + agg_ref[1].astype(jnp.float32))
    up = _ssp(
        jnp.dot(agg, wu_ref[...], preferred_element_type=jnp.float32) +
        bu_ref[...])
    h = h_ref[...] + up
    h_out_ref[...] = h
    hm_out_ref[...] = jnp.dot(h, wn_ref[...],
                              preferred_element_type=jnp.float32)


def _tc_update(h, agg2, wu, bu, wnext, bn):
    grid = (N // bn,)
    return pl.pallas_call(
        _upd_body,
        grid=grid,
        in_specs=[
            pl.BlockSpec((bn, H), lambda n: (n, 0)),
            pl.BlockSpec((2, bn, H), lambda n: (0, n, 0)),
            pl.BlockSpec((H, H), lambda n: (0, 0)),
            pl.BlockSpec((1, H), lambda n: (0, 0)),
            pl.BlockSpec((H, H), lambda n: (0, 0)),
        ],
        out_specs=[
            pl.BlockSpec((bn, H), lambda n: (n, 0)),
            pl.BlockSpec((bn, H), lambda n: (n, 0)),
        ],
        out_shape=[
            jax.ShapeDtypeStruct((N, H), jnp.float32),
            jax.ShapeDtypeStruct((N, H), jnp.float32),
        ],
    )(h, agg2, wu, bu.reshape(1, H), wnext)


def _final_body(h_ref, agg_ref, wu_ref, bu_ref, w1_ref, b1_ref, batch_ref,
                out_ref):
    agg = (agg_ref[0].astype(jnp.float32) + agg_ref[1].astype(jnp.float32))
    up = _ssp(
        jnp.dot(agg, wu_ref[...], preferred_element_type=jnp.float32) +
        bu_ref[...])
    h = h_ref[...] + up
    z = _ssp(
        jnp.dot(h, w1_ref[...], preferred_element_type=jnp.float32) +
        b1_ref[...])
    b = batch_ref[0, 0]
    onehot = (b[None, :] == lax.broadcasted_iota(jnp.int32, (NG, 1),
                                                 0)).astype(jnp.float32)
    part = jnp.dot(onehot, z, preferred_element_type=jnp.float32)

    @pl.when(pl.program_id(0) == 0)
    def _():
        out_ref[...] = jnp.zeros_like(out_ref)

    out_ref[...] += part


def _tc_final(h, agg2, wu, bu, w1, b1, batch, bn):
    grid = (N // bn,)
    return pl.pallas_call(
        _final_body,
        grid=grid,
        in_specs=[
            pl.BlockSpec((bn, H), lambda n: (n, 0)),
            pl.BlockSpec((2, bn, H), lambda n: (0, n, 0)),
            pl.BlockSpec((H, H), lambda n: (0, 0)),
            pl.BlockSpec((1, H), lambda n: (0, 0)),
            pl.BlockSpec((H, L_OUT), lambda n: (0, 0)),
            pl.BlockSpec((1, L_OUT), lambda n: (0, 0)),
            pl.BlockSpec((1, 1, bn), lambda n: (n, 0, 0)),
        ],
        out_specs=pl.BlockSpec((NG, L_OUT), lambda n: (0, 0)),
        out_shape=jax.ShapeDtypeStruct((NG, L_OUT), jnp.float32),
    )(h, agg2, wu, bu.reshape(1, H), w1, b1.reshape(1, L_OUT),
      batch.reshape(N // bn, 1, bn))


def kernel(x, pos, batch, edge_index_3rd, W0, b0, Wg1, bg1, Wg2, bg2, Wmsg,
           Wupd, bupd, W1, b1):
    pos_flat = jnp.pad(pos, ((0, 0), (0, 1))).reshape(4 * N)
    ei = jnp.pad(edge_index_3rd.astype(jnp.int32),
                 ((0, 0), (0, E_PAD - E)))
    idx_flat = ei.reshape(4 * E_PAD)
    off_d = jnp.linspace(0.0, CUTOFF, 50)
    off_a = jnp.linspace(0.0, math.pi, 6)
    off_t = jnp.linspace(-math.pi, math.pi, 12)
    coef_d = jnp.full((50,), -0.5 / (CUTOFF / 49.0) ** 2)
    coef_a = jnp.full((6,), -0.5 / (math.pi / 5.0) ** 2)
    coef_t = jnp.full((12,), -0.5 / (2.0 * math.pi / 11.0) ** 2)
    pad0 = jnp.zeros((128 - G_TOTAL,))
    offs = jnp.concatenate([off_d, off_a, off_t, pad0]).astype(
        jnp.float32).reshape(1, 128)
    coefs = jnp.concatenate([coef_d, coef_a, coef_t, pad0]).astype(
        jnp.float32).reshape(1, 128)
    wg1p = jnp.pad(Wg1, ((0, 0), (0, 128 - G_TOTAL), (0, 0)))

    p4 = _sc_gather_pos(pos_flat, idx_flat).reshape(3, 4, E_PAD)

    ii = ei[0]
    jj = ei[1]

    h, hm = _tc_h0(x, W0, b0, Wmsg[0], bn=2000)
    for t in range(3):
        g_t = _tc_gates_t(p4, wg1p[t], bg1[t], Wg2[t], bg2[t], offs, coefs,
                          be=2048)
        agg2 = _sc_gather_mul_scatter(hm, g_t, jj, ii)
        if t < 2:
            h, hm = _tc_update(h, agg2, Wupd[t], bupd[t], Wmsg[t + 1],
                               bn=2000)
        else:
            out = _tc_final(h, agg2, Wupd[t], bupd[t], W1, b1, batch,
                            bn=2000)
    return out
